# Initial kernel scaffold; baseline (speedup 1.0000x reference)
import jax
import jax.numpy as jnp
from jax.experimental import pallas as pl


def kernel(position_ids, weight):
    raise NotImplementedError("write your pallas kernel here")


# speedup vs baseline: 1.9854x; 1.9854x over previous
import functools

import jax
import jax.numpy as jnp
from jax import lax
from jax.experimental import pallas as pl
from jax.experimental.pallas import tpu as pltpu
from jax.experimental.pallas import tpu_sc as plsc

_D = 1024
_NW = 32
_NC = 2
_CH = 32


def _emb_body(idx_hbm, table_hbm, out_hbm, idx_v, rows_v, gsem):
    bpw = idx_hbm.shape[0] // _NW
    nchunk = bpw // _CH
    wid = lax.axis_index("s") * _NC + lax.axis_index("c")
    base = wid * bpw

    pltpu.sync_copy(idx_hbm.at[pl.ds(base, bpw)], idx_v)

    def chunk(c, carry):
        off = c * _CH
        pltpu.async_copy(
            table_hbm.at[idx_v.at[pl.ds(off, _CH)]], rows_v, gsem
        ).wait()
        pltpu.sync_copy(rows_v, out_hbm.at[pl.ds(base + off, _CH)])
        return carry

    lax.fori_loop(0, nchunk, chunk, 0)


def kernel(position_ids, weight):
    batch, seq = position_ids.shape
    b = batch * seq
    idx = position_ids.reshape(b).astype(jnp.int32)

    mesh = plsc.VectorSubcoreMesh(core_axis_name="c", subcore_axis_name="s")
    bpw = b // _NW

    run = functools.partial(
        pl.kernel,
        mesh=mesh,
        out_type=jax.ShapeDtypeStruct((b, _D), jnp.float32),
        scratch_types=[
            pltpu.VMEM((bpw,), jnp.int32),
            pltpu.VMEM((_CH, _D), jnp.float32),
            pltpu.SemaphoreType.DMA,
        ],
    )(_emb_body)

    out = run(idx, weight)
    return out.reshape(batch, seq, _D)


# speedup vs baseline: 2.3656x; 1.1915x over previous
import functools

import jax
import jax.numpy as jnp
from jax import lax
from jax.experimental import pallas as pl
from jax.experimental.pallas import tpu as pltpu
from jax.experimental.pallas import tpu_sc as plsc

_D = 1024
_NW = 32
_NC = 2
_CH = 32


_NBUF = 2


def _emb_body(idx_hbm, table_hbm, out_hbm, idx_v, rows_v, g0, g1, s0, s1):
    gsems = (g0, g1)
    ssems = (s0, s1)
    bpw = idx_hbm.shape[0] // _NW
    nchunk = bpw // _CH
    ngroup = nchunk // _NBUF
    wid = lax.axis_index("s") * _NC + lax.axis_index("c")
    base = wid * bpw

    pltpu.sync_copy(idx_hbm.at[pl.ds(base, bpw)], idx_v)

    def gather(c, bf):
        return pltpu.make_async_copy(
            table_hbm.at[idx_v.at[pl.ds(c * _CH, _CH)]], rows_v.at[bf],
            gsems[bf])

    def scatter(c, bf):
        return pltpu.make_async_copy(
            rows_v.at[bf], out_hbm.at[pl.ds(base + c * _CH, _CH)], ssems[bf])

    for bf in range(_NBUF):
        gather(bf, bf).start()

    def group(g, carry):
        for bf in range(_NBUF):
            c = g * _NBUF + bf
            gather(c, bf).wait()
            scatter(c, bf).start()
            scatter(c, bf).wait()
            gather(c + _NBUF, bf).start()
        return carry

    lax.fori_loop(0, ngroup - 1, group, 0)

    for bf in range(_NBUF):
        c = (ngroup - 1) * _NBUF + bf
        gather(c, bf).wait()
        scatter(c, bf).start()
        scatter(c, bf).wait()


def kernel(position_ids, weight):
    batch, seq = position_ids.shape
    b = batch * seq
    idx = position_ids.reshape(b).astype(jnp.int32)

    mesh = plsc.VectorSubcoreMesh(core_axis_name="c", subcore_axis_name="s")
    bpw = b // _NW

    run = functools.partial(
        pl.kernel,
        mesh=mesh,
        out_type=jax.ShapeDtypeStruct((b, _D), jnp.float32),
        scratch_types=[
            pltpu.VMEM((bpw,), jnp.int32),
            pltpu.VMEM((_NBUF, _CH, _D), jnp.float32),
            pltpu.SemaphoreType.DMA,
            pltpu.SemaphoreType.DMA,
            pltpu.SemaphoreType.DMA,
            pltpu.SemaphoreType.DMA,
        ],
    )(_emb_body)

    out = run(idx, weight)
    return out.reshape(batch, seq, _D)


# speedup vs baseline: 2.3846x; 1.0080x over previous
import functools

import jax
import jax.numpy as jnp
from jax import lax
from jax.experimental import pallas as pl
from jax.experimental.pallas import tpu as pltpu
from jax.experimental.pallas import tpu_sc as plsc

_D = 1024
_NW = 32
_NC = 2
_CH = 16


_NBUF = 4


def _emb_body(idx_hbm, table_hbm, out_hbm, idx_v, rows_v,
              g0, g1, g2, g3, s0, s1, s2, s3):
    gsems = (g0, g1, g2, g3)
    ssems = (s0, s1, s2, s3)
    bpw = idx_hbm.shape[0] // _NW
    nchunk = bpw // _CH
    ngroup = nchunk // _NBUF
    wid = lax.axis_index("s") * _NC + lax.axis_index("c")
    base = wid * bpw

    pltpu.sync_copy(idx_hbm.at[pl.ds(base, bpw)], idx_v)

    def gather(c, bf):
        return pltpu.make_async_copy(
            table_hbm.at[idx_v.at[pl.ds(c * _CH, _CH)]], rows_v.at[bf],
            gsems[bf])

    def scatter(c, bf):
        return pltpu.make_async_copy(
            rows_v.at[bf], out_hbm.at[pl.ds(base + c * _CH, _CH)], ssems[bf])

    for bf in range(_NBUF):
        gather(bf, bf).start()

    def group(g, carry):
        for bf in range(_NBUF):
            c = g * _NBUF + bf
            gather(c, bf).wait()
            scatter(c, bf).start()
            scatter(c, bf).wait()
            gather(c + _NBUF, bf).start()
        return carry

    lax.fori_loop(0, ngroup - 1, group, 0)

    for bf in range(_NBUF):
        c = (ngroup - 1) * _NBUF + bf
        gather(c, bf).wait()
        scatter(c, bf).start()
        scatter(c, bf).wait()


def kernel(position_ids, weight):
    batch, seq = position_ids.shape
    b = batch * seq
    idx = position_ids.reshape(b).astype(jnp.int32)

    mesh = plsc.VectorSubcoreMesh(core_axis_name="c", subcore_axis_name="s")
    bpw = b // _NW

    run = functools.partial(
        pl.kernel,
        mesh=mesh,
        out_type=jax.ShapeDtypeStruct((b, _D), jnp.float32),
        scratch_types=[
            pltpu.VMEM((bpw,), jnp.int32),
            pltpu.VMEM((_NBUF, _CH, _D), jnp.float32),
        ] + [pltpu.SemaphoreType.DMA] * 8,
    )(_emb_body)

    out = run(idx, weight)
    return out.reshape(batch, seq, _D)
